# baseline (speedup 1.0000x reference)
import functools

import jax
import jax.numpy as jnp
from jax import lax
from jax.experimental import pallas as pl
from jax.experimental.pallas import tpu as pltpu
from jax.experimental.pallas import tpu_sc as plsc

S = 2048
H = 2048
E = 8
K = 2
I = 1408
IS = 2 * I
SK = S * K
T = 256
G = SK // T + E - 1
R = G * T

_C = 512
F32 = jnp.float32
I32 = jnp.int32
BF16 = jnp.bfloat16


def _router_body(x_ref, wg_ref, idx_ref, w_ref, pos_ref, meta_ref):
    x = x_ref[...]
    logits = lax.dot_general(x, wg_ref[...], (((1,), (1,)), ((), ())),
                             preferred_element_type=F32)
    m = jnp.max(logits, axis=1, keepdims=True)
    p = jnp.exp(logits - m)
    scores = p / jnp.sum(p, axis=1, keepdims=True)

    i8 = lax.broadcasted_iota(I32, (S, E), 1).astype(F32)
    w0 = jnp.max(scores, axis=1, keepdims=True)
    idx0 = jnp.min(jnp.where(scores == w0, i8, float(E)), axis=1,
                   keepdims=True)
    masked = jnp.where(i8 == idx0, -1.0, scores)
    w1 = jnp.max(masked, axis=1, keepdims=True)
    idx1 = jnp.min(jnp.where(masked == w1, i8, float(E)), axis=1,
                   keepdims=True)

    idx_ref[...] = jnp.concatenate([idx0.astype(I32), idx1.astype(I32)],
                                   axis=1)
    w_cat = jnp.concatenate([w0, w1], axis=0)
    w_ref[...] = w_cat * jnp.ones((1, 16), F32)

    oh = jnp.concatenate([(i8 == idx0).astype(F32),
                          (i8 == idx1).astype(F32)], axis=0)

    tri = (lax.broadcasted_iota(I32, (_C, _C), 0) >=
           lax.broadcasted_iota(I32, (_C, _C), 1)).astype(F32)
    carry = jnp.zeros((1, E), F32)
    chunks = []
    for c in range(SK // _C):
        seg = oh[c * _C:(c + 1) * _C]
        incl_c = lax.dot_general(tri, seg, (((1,), (0,)), ((), ())),
                                 preferred_element_type=F32) + carry
        chunks.append(incl_c)
        carry = incl_c[_C - 1:_C, :]
    incl = jnp.concatenate(chunks, axis=0)
    rank = incl - oh

    counts = incl[SK - 1:SK, :]
    ci = counts.astype(I32)
    pc = jnp.bitwise_and(ci + (T - 1), -T)
    pcf = pc.astype(F32)
    upper = (lax.broadcasted_iota(I32, (E, E), 0) <
             lax.broadcasted_iota(I32, (E, E), 1)).astype(F32)
    aligned = lax.dot_general(pcf, upper, (((1,), (0,)), ((), ())),
                              preferred_element_type=F32)
    posf = jnp.sum(oh * (aligned + rank), axis=1, keepdims=True)
    pos_ref[...] = posf.astype(I32)

    ends = aligned + pcf
    gvals = lax.broadcasted_iota(I32, (G, 1), 0).astype(F32) * float(T)
    be = jnp.sum((gvals >= ends).astype(F32), axis=1, keepdims=True)
    eid = lax.broadcasted_iota(I32, (1, E), 1).astype(F32)
    last_e = jnp.max(jnp.where(pcf > 0, eid, 0.0), axis=1, keepdims=True)
    bec = jnp.minimum(be, last_e)
    used = jnp.sum(pcf, axis=1, keepdims=True) * (1.0 / T)
    meta_ref[...] = jnp.concatenate([used, bec], axis=0).astype(I32)


def _router(x, w_gate):
    return pl.pallas_call(
        _router_body,
        out_shape=[
            jax.ShapeDtypeStruct((S, K), I32),
            jax.ShapeDtypeStruct((SK, 16), F32),
            jax.ShapeDtypeStruct((SK, 1), I32),
            jax.ShapeDtypeStruct((G + 1, 1), I32),
        ],
    )(x, w_gate)


_NW = 32
_PPW = SK // _NW
_PB = 16


def _builder(x, pos2):
    mesh = plsc.VectorSubcoreMesh(core_axis_name="c", subcore_axis_name="s")
    nb = _PPW // _PB

    @functools.partial(
        pl.kernel,
        mesh=mesh,
        out_type=jax.ShapeDtypeStruct((R, H), F32),
        scratch_types=[
            pltpu.VMEM((nb, _PB), I32),
            pltpu.VMEM((2, _PB), I32),
            pltpu.VMEM((2, _PB, H), F32),
            pltpu.SemaphoreType.DMA,
            pltpu.SemaphoreType.DMA,
            pltpu.SemaphoreType.DMA,
            pltpu.SemaphoreType.DMA,
        ],
    )
    def body(x_hbm, pos_hbm, xs_hbm, posv, tokv, rows,
             gsem0, gsem1, ssem0, ssem1):
        wid = lax.axis_index("s") * 2 + lax.axis_index("c")
        sub = jnp.where(wid >= _NW // 2, S, 0)
        base0 = wid * _PPW
        pltpu.sync_copy(pos_hbm.at[pl.ds(wid * nb, nb)], posv)
        gsems = (gsem0, gsem1)
        ssems = (ssem0, ssem1)

        def start_gather(b):
            sl = b % 2
            for h in range(_PB // 16):
                tokv[sl, pl.ds(h * 16, 16)] = (
                    base0 - sub + b * _PB + h * 16
                    + lax.broadcasted_iota(I32, (16,), 0))
            return pltpu.async_copy(x_hbm.at[tokv.at[sl]], rows.at[sl],
                                    gsems[sl])

        ghand = [start_gather(0), None]
        shand = [None, None]
        for b in range(nb):
            sl = b % 2
            ghand[sl].wait()
            if b + 1 < nb:
                osl = (b + 1) % 2
                if shand[osl] is not None:
                    shand[osl].wait()
                ghand[osl] = start_gather(b + 1)
            shand[sl] = pltpu.async_copy(rows.at[sl], xs_hbm.at[posv.at[b]],
                                         ssems[sl])
        for hnd in shand:
            if hnd is not None:
                hnd.wait()

    return body(x, pos2)


def _gup_body(m_ref, xs_ref, wg_ref, wu_ref, h_ref):
    g = pl.program_id(0)

    @pl.when(g < m_ref[0])
    def _():
        xb = xs_ref[...].astype(BF16)
        gate = lax.dot_general(xb, wg_ref[0].astype(BF16),
                               (((1,), (1,)), ((), ())),
                               preferred_element_type=F32)
        up = lax.dot_general(xb, wu_ref[0].astype(BF16),
                             (((1,), (1,)), ((), ())),
                             preferred_element_type=F32)
        h_ref[...] = jax.nn.silu(gate) * up


def _down_body(m_ref, h_ref, wd_ref, o_ref):
    g = pl.program_id(0)

    @pl.when(g < m_ref[0])
    def _():
        o_ref[...] = lax.dot_general(h_ref[...].astype(BF16),
                                     wd_ref[0].astype(BF16),
                                     (((1,), (1,)), ((), ())),
                                     preferred_element_type=F32)


def _grouped(meta, xs, wg_e, wu_e, wd_e):
    gup_spec = pltpu.PrefetchScalarGridSpec(
        num_scalar_prefetch=1,
        grid=(G,),
        in_specs=[
            pl.BlockSpec((T, H), lambda g, m: (jnp.minimum(g, m[0] - 1), 0)),
            pl.BlockSpec((1, I, H), lambda g, m: (m[1 + g], 0, 0)),
            pl.BlockSpec((1, I, H), lambda g, m: (m[1 + g], 0, 0)),
        ],
        out_specs=pl.BlockSpec((T, I), lambda g, m: (g, 0)),
    )
    h = pl.pallas_call(
        _gup_body,
        grid_spec=gup_spec,
        out_shape=jax.ShapeDtypeStruct((R, I), F32),
    )(meta, xs, wg_e, wu_e)
    down_spec = pltpu.PrefetchScalarGridSpec(
        num_scalar_prefetch=1,
        grid=(G,),
        in_specs=[
            pl.BlockSpec((T, I), lambda g, m: (jnp.minimum(g, m[0] - 1), 0)),
            pl.BlockSpec((1, H, I), lambda g, m: (m[1 + g], 0, 0)),
        ],
        out_specs=pl.BlockSpec((T, H), lambda g, m: (g, 0)),
    )
    return pl.pallas_call(
        _down_body,
        grid_spec=down_spec,
        out_shape=jax.ShapeDtypeStruct((R, H), F32),
    )(meta, h, wd_e)


def _ygather(ys, pos2):
    mesh = plsc.VectorSubcoreMesh(core_axis_name="c", subcore_axis_name="s")
    nb = _PPW // _PB

    @functools.partial(
        pl.kernel,
        mesh=mesh,
        out_type=jax.ShapeDtypeStruct((SK, H), F32),
        scratch_types=[
            pltpu.VMEM((nb, _PB), I32),
            pltpu.VMEM((2, _PB, H), F32),
            pltpu.SemaphoreType.DMA,
            pltpu.SemaphoreType.DMA,
            pltpu.SemaphoreType.DMA,
            pltpu.SemaphoreType.DMA,
        ],
    )
    def body(ys_hbm, pos_hbm, yg_hbm, posv, rows,
             gsem0, gsem1, wsem0, wsem1):
        wid = lax.axis_index("s") * 2 + lax.axis_index("c")
        base0 = wid * _PPW
        pltpu.sync_copy(pos_hbm.at[pl.ds(wid * nb, nb)], posv)
        gsems = (gsem0, gsem1)
        wsems = (wsem0, wsem1)

        def start_gather(b):
            sl = b % 2
            return pltpu.async_copy(ys_hbm.at[posv.at[b]], rows.at[sl],
                                    gsems[sl])

        ghand = [start_gather(0), None]
        whand = [None, None]
        for b in range(nb):
            sl = b % 2
            ghand[sl].wait()
            if b + 1 < nb:
                osl = (b + 1) % 2
                if whand[osl] is not None:
                    whand[osl].wait()
                ghand[osl] = start_gather(b + 1)
            whand[sl] = pltpu.async_copy(
                rows.at[sl], yg_hbm.at[pl.ds(base0 + b * _PB, _PB)],
                wsems[sl])
        for hnd in whand:
            if hnd is not None:
                hnd.wait()

    return body(ys, pos2)


def _sgup_body(x_ref, wg_ref, wu_ref, h_ref):
    xb = x_ref[...].astype(BF16)
    gate = lax.dot_general(xb, wg_ref[...].astype(BF16),
                           (((1,), (1,)), ((), ())),
                           preferred_element_type=F32)
    up = lax.dot_general(xb, wu_ref[...].astype(BF16),
                         (((1,), (1,)), ((), ())),
                         preferred_element_type=F32)
    h_ref[...] = jax.nn.silu(gate) * up


def _sdown_body(h_ref, wd_ref, o_ref):
    o_ref[...] = lax.dot_general(h_ref[...].astype(BF16),
                                 wd_ref[...].astype(BF16),
                                 (((1,), (1,)), ((), ())),
                                 preferred_element_type=F32)


def _addmix_body(sd_ref, y0_ref, y1_ref, w0_ref, w1_ref, o_ref):
    o_ref[...] = (sd_ref[...] + w0_ref[:, 0:1] * y0_ref[...]
                  + w1_ref[:, 0:1] * y1_ref[...])


_TS = 256


def _sgup(x, wg_s, wu_s):
    return pl.pallas_call(
        _sgup_body,
        grid=(S // _TS,),
        in_specs=[
            pl.BlockSpec((_TS, H), lambda g: (g, 0)),
            pl.BlockSpec((IS, H), lambda g: (0, 0)),
            pl.BlockSpec((IS, H), lambda g: (0, 0)),
        ],
        out_specs=pl.BlockSpec((_TS, IS), lambda g: (g, 0)),
        out_shape=jax.ShapeDtypeStruct((S, IS), F32),
    )(x, wg_s, wu_s)


def _sdown(hs, wd_s):
    return pl.pallas_call(
        _sdown_body,
        grid=(S // _TS,),
        in_specs=[
            pl.BlockSpec((_TS, IS), lambda g: (g, 0)),
            pl.BlockSpec((H, IS), lambda g: (0, 0)),
        ],
        out_specs=pl.BlockSpec((_TS, H), lambda g: (g, 0)),
        out_shape=jax.ShapeDtypeStruct((S, H), F32),
    )(hs, wd_s)


def _addmix(sd, ygath, wsplat):
    nts = S // _TS
    return pl.pallas_call(
        _addmix_body,
        grid=(nts,),
        in_specs=[
            pl.BlockSpec((_TS, H), lambda g: (g, 0)),
            pl.BlockSpec((_TS, H), lambda g: (g, 0)),
            pl.BlockSpec((_TS, H), lambda g, n=nts: (n + g, 0)),
            pl.BlockSpec((_TS, 16), lambda g: (g, 0)),
            pl.BlockSpec((_TS, 16), lambda g, n=nts: (n + g, 0)),
        ],
        out_specs=pl.BlockSpec((_TS, H), lambda g: (g, 0)),
        out_shape=jax.ShapeDtypeStruct((S, H), F32),
    )(sd, ygath, ygath, wsplat, wsplat)


def kernel(hidden_states, W_gate, Wg_e, Wu_e, Wd_e, Wg_s, Wu_s, Wd_s):
    x = hidden_states.reshape(S, H)
    o_idx, o_w, o_pos, o_meta = _router(x, W_gate)
    meta = o_meta.reshape(G + 1)
    pos2 = o_pos.reshape(SK // _PB, _PB)
    xs = _builder(x, pos2)
    hs = _sgup(x, Wg_s, Wu_s)
    ys = _grouped(meta, xs, Wg_e, Wu_e, Wd_e)
    ygath = _ygather(ys, pos2)
    sd = _sdown(hs, Wd_s)
    out = _addmix(sd, ygath, o_w)
    return out.reshape(1, S, H), o_idx.reshape(1, S, K)


# speedup vs baseline: 1.0682x; 1.0682x over previous
import functools

import jax
import jax.numpy as jnp
from jax import lax
from jax.experimental import pallas as pl
from jax.experimental.pallas import tpu as pltpu
from jax.experimental.pallas import tpu_sc as plsc

S = 2048
H = 2048
E = 8
K = 2
I = 1408
IS = 2 * I
SK = S * K
T = 256
G = SK // T + E - 1
R = G * T

_C = 512
F32 = jnp.float32
I32 = jnp.int32
BF16 = jnp.bfloat16


def _router_body(x_ref, wg_ref, idx_ref, w_ref, pos_ref, meta_ref):
    x = x_ref[...]
    logits = lax.dot_general(x, wg_ref[...], (((1,), (1,)), ((), ())),
                             preferred_element_type=F32)
    m = jnp.max(logits, axis=1, keepdims=True)
    p = jnp.exp(logits - m)
    scores = p / jnp.sum(p, axis=1, keepdims=True)

    i8 = lax.broadcasted_iota(I32, (S, E), 1).astype(F32)
    w0 = jnp.max(scores, axis=1, keepdims=True)
    idx0 = jnp.min(jnp.where(scores == w0, i8, float(E)), axis=1,
                   keepdims=True)
    masked = jnp.where(i8 == idx0, -1.0, scores)
    w1 = jnp.max(masked, axis=1, keepdims=True)
    idx1 = jnp.min(jnp.where(masked == w1, i8, float(E)), axis=1,
                   keepdims=True)

    idx_ref[...] = jnp.concatenate([idx0.astype(I32), idx1.astype(I32)],
                                   axis=1)
    w_cat = jnp.concatenate([w0, w1], axis=0)
    w_ref[...] = w_cat * jnp.ones((1, 16), F32)

    oh = jnp.concatenate([(i8 == idx0).astype(F32),
                          (i8 == idx1).astype(F32)], axis=0)

    tri = (lax.broadcasted_iota(I32, (_C, _C), 0) >=
           lax.broadcasted_iota(I32, (_C, _C), 1)).astype(F32)
    carry = jnp.zeros((1, E), F32)
    chunks = []
    for c in range(SK // _C):
        seg = oh[c * _C:(c + 1) * _C]
        incl_c = lax.dot_general(tri, seg, (((1,), (0,)), ((), ())),
                                 preferred_element_type=F32) + carry
        chunks.append(incl_c)
        carry = incl_c[_C - 1:_C, :]
    incl = jnp.concatenate(chunks, axis=0)
    rank = incl - oh

    counts = incl[SK - 1:SK, :]
    ci = counts.astype(I32)
    pc = jnp.bitwise_and(ci + (T - 1), -T)
    pcf = pc.astype(F32)
    upper = (lax.broadcasted_iota(I32, (E, E), 0) <
             lax.broadcasted_iota(I32, (E, E), 1)).astype(F32)
    aligned = lax.dot_general(pcf, upper, (((1,), (0,)), ((), ())),
                              preferred_element_type=F32)
    posf = jnp.sum(oh * (aligned + rank), axis=1, keepdims=True)
    pos_ref[...] = posf.astype(I32)

    ends = aligned + pcf
    gvals = lax.broadcasted_iota(I32, (G, 1), 0).astype(F32) * float(T)
    be = jnp.sum((gvals >= ends).astype(F32), axis=1, keepdims=True)
    eid = lax.broadcasted_iota(I32, (1, E), 1).astype(F32)
    last_e = jnp.max(jnp.where(pcf > 0, eid, 0.0), axis=1, keepdims=True)
    bec = jnp.minimum(be, last_e)
    used = jnp.sum(pcf, axis=1, keepdims=True) * (1.0 / T)
    meta_ref[...] = jnp.concatenate([used, bec], axis=0).astype(I32)


def _router(x, w_gate):
    return pl.pallas_call(
        _router_body,
        out_shape=[
            jax.ShapeDtypeStruct((S, K), I32),
            jax.ShapeDtypeStruct((SK, 16), F32),
            jax.ShapeDtypeStruct((SK, 1), I32),
            jax.ShapeDtypeStruct((G + 1, 1), I32),
        ],
    )(x, w_gate)


_NW = 32
_PPW = SK // _NW
_PB = 16


def _builder(x, pos2):
    mesh = plsc.VectorSubcoreMesh(core_axis_name="c", subcore_axis_name="s")
    nb = _PPW // _PB

    @functools.partial(
        pl.kernel,
        mesh=mesh,
        out_type=jax.ShapeDtypeStruct((R, H), F32),
        scratch_types=[
            pltpu.VMEM((nb, _PB), I32),
            pltpu.VMEM((2, _PB), I32),
            pltpu.VMEM((2, _PB, H), F32),
            pltpu.SemaphoreType.DMA,
            pltpu.SemaphoreType.DMA,
            pltpu.SemaphoreType.DMA,
            pltpu.SemaphoreType.DMA,
        ],
    )
    def body(x_hbm, pos_hbm, xs_hbm, posv, tokv, rows,
             gsem0, gsem1, ssem0, ssem1):
        wid = lax.axis_index("s") * 2 + lax.axis_index("c")
        sub = jnp.where(wid >= _NW // 2, S, 0)
        base0 = wid * _PPW
        pltpu.sync_copy(pos_hbm.at[pl.ds(wid * nb, nb)], posv)
        gsems = (gsem0, gsem1)
        ssems = (ssem0, ssem1)

        def start_gather(b):
            sl = b % 2
            for h in range(_PB // 16):
                tokv[sl, pl.ds(h * 16, 16)] = (
                    base0 - sub + b * _PB + h * 16
                    + lax.broadcasted_iota(I32, (16,), 0))
            return pltpu.async_copy(x_hbm.at[tokv.at[sl]], rows.at[sl],
                                    gsems[sl])

        ghand = [start_gather(0), None]
        shand = [None, None]
        for b in range(nb):
            sl = b % 2
            ghand[sl].wait()
            if b + 1 < nb:
                osl = (b + 1) % 2
                if shand[osl] is not None:
                    shand[osl].wait()
                ghand[osl] = start_gather(b + 1)
            shand[sl] = pltpu.async_copy(rows.at[sl], xs_hbm.at[posv.at[b]],
                                         ssems[sl])
        for hnd in shand:
            if hnd is not None:
                hnd.wait()

    return body(x, pos2)


def _gup_body(m_ref, xs_ref, wg_ref, wu_ref, h_ref):
    g = pl.program_id(0)

    @pl.when(g < m_ref[0])
    def _():
        xb = xs_ref[...].astype(BF16)
        gate = lax.dot_general(xb, wg_ref[0].astype(BF16),
                               (((1,), (1,)), ((), ())),
                               preferred_element_type=F32)
        up = lax.dot_general(xb, wu_ref[0].astype(BF16),
                             (((1,), (1,)), ((), ())),
                             preferred_element_type=F32)
        h_ref[...] = jax.nn.silu(gate) * up


def _down_body(m_ref, h_ref, wd_ref, o_ref):
    g = pl.program_id(0)

    @pl.when(g < m_ref[0])
    def _():
        o_ref[...] = lax.dot_general(h_ref[...].astype(BF16),
                                     wd_ref[0].astype(BF16),
                                     (((1,), (1,)), ((), ())),
                                     preferred_element_type=F32)


def _grouped(meta, xs, wg_e, wu_e, wd_e):
    gup_spec = pltpu.PrefetchScalarGridSpec(
        num_scalar_prefetch=1,
        grid=(G,),
        in_specs=[
            pl.BlockSpec((T, H), lambda g, m: (jnp.minimum(g, m[0] - 1), 0)),
            pl.BlockSpec((1, I, H), lambda g, m: (m[1 + g], 0, 0)),
            pl.BlockSpec((1, I, H), lambda g, m: (m[1 + g], 0, 0)),
        ],
        out_specs=pl.BlockSpec((T, I), lambda g, m: (g, 0)),
    )
    h = pl.pallas_call(
        _gup_body,
        grid_spec=gup_spec,
        out_shape=jax.ShapeDtypeStruct((R, I), F32),
    )(meta, xs, wg_e, wu_e)
    down_spec = pltpu.PrefetchScalarGridSpec(
        num_scalar_prefetch=1,
        grid=(G,),
        in_specs=[
            pl.BlockSpec((T, I), lambda g, m: (jnp.minimum(g, m[0] - 1), 0)),
            pl.BlockSpec((1, H, I), lambda g, m: (m[1 + g], 0, 0)),
        ],
        out_specs=pl.BlockSpec((T, H), lambda g, m: (g, 0)),
    )
    return pl.pallas_call(
        _down_body,
        grid_spec=down_spec,
        out_shape=jax.ShapeDtypeStruct((R, H), F32),
    )(meta, h, wd_e)


_TPW = S // _NW
_TB = 8
_CU = 4


def _combine(ys, pos, wsplat):
    mesh = plsc.VectorSubcoreMesh(core_axis_name="c", subcore_axis_name="s")
    nb = _TPW // _TB

    @functools.partial(
        pl.kernel,
        mesh=mesh,
        out_type=jax.ShapeDtypeStruct((S, H), F32),
        scratch_types=[
            pltpu.VMEM((2, 2 * _TB), I32),
            pltpu.VMEM((2 * _TB, 16), F32),
            pltpu.VMEM((2, 2 * _TB, H), F32),
            pltpu.VMEM((2, _TB, H), F32),
            pltpu.SemaphoreType.DMA,
            pltpu.SemaphoreType.DMA,
            pltpu.SemaphoreType.DMA,
            pltpu.SemaphoreType.DMA,
        ],
    )
    def body(ys_hbm, pos_hbm, w_hbm, out_hbm, idxv, wv, yrows, orows,
             gsem0, gsem1, wsem0, wsem1):
        wid = lax.axis_index("s") * 2 + lax.axis_index("c")
        gsems = (gsem0, gsem1)
        wsems = (wsem0, wsem1)

        def start_gather(b):
            sl = b % 2
            t0 = wid * _TPW + b * _TB
            pltpu.sync_copy(pos_hbm.at[pl.ds(t0, _TB)],
                            idxv.at[sl, pl.ds(0, _TB)])
            pltpu.sync_copy(pos_hbm.at[pl.ds(S + t0, _TB)],
                            idxv.at[sl, pl.ds(_TB, _TB)])
            return pltpu.async_copy(ys_hbm.at[idxv.at[sl]], yrows.at[sl],
                                    gsems[sl])

        ghand = [start_gather(0), None]
        whand = [None, None]
        for b in range(nb):
            sl = b % 2
            t0 = wid * _TPW + b * _TB
            if b + 1 < nb:
                ghand[(b + 1) % 2] = start_gather(b + 1)
            pltpu.sync_copy(w_hbm.at[pl.ds(t0, _TB)], wv.at[pl.ds(0, _TB)])
            pltpu.sync_copy(w_hbm.at[pl.ds(S + t0, _TB)],
                            wv.at[pl.ds(_TB, _TB)])
            wa = [wv[j, pl.ds(0, 16)] for j in range(_TB)]
            wb = [wv[_TB + j, pl.ds(0, 16)] for j in range(_TB)]
            ghand[sl].wait()
            if whand[sl] is not None:
                whand[sl].wait()

            def cbody(c, _):
                for cc in range(_CU):
                    sl2 = pl.ds((c * _CU + cc) * 16, 16)
                    for j in range(_TB):
                        orows[sl, j, sl2] = (wa[j] * yrows[sl, j, sl2]
                                             + wb[j] * yrows[sl, _TB + j, sl2])
                return 0
            lax.fori_loop(0, H // (16 * _CU), cbody, 0)
            whand[sl] = pltpu.async_copy(orows.at[sl],
                                         out_hbm.at[pl.ds(t0, _TB)],
                                         wsems[sl])
        for hnd in whand:
            if hnd is not None:
                hnd.wait()

    return body(ys, pos, wsplat)


def _sgup_body(x_ref, wg_ref, wu_ref, h_ref):
    xb = x_ref[...].astype(BF16)
    gate = lax.dot_general(xb, wg_ref[...].astype(BF16),
                           (((1,), (1,)), ((), ())),
                           preferred_element_type=F32)
    up = lax.dot_general(xb, wu_ref[...].astype(BF16),
                         (((1,), (1,)), ((), ())),
                         preferred_element_type=F32)
    h_ref[...] = jax.nn.silu(gate) * up


def _sdown_body(h_ref, wd_ref, r_ref, o_ref):
    o_ref[...] = (lax.dot_general(h_ref[...].astype(BF16),
                                  wd_ref[...].astype(BF16),
                                  (((1,), (1,)), ((), ())),
                                  preferred_element_type=F32)
                  + r_ref[...])


_TS = 256


def _sgup(x, wg_s, wu_s):
    return pl.pallas_call(
        _sgup_body,
        grid=(S // _TS,),
        in_specs=[
            pl.BlockSpec((_TS, H), lambda g: (g, 0)),
            pl.BlockSpec((IS, H), lambda g: (0, 0)),
            pl.BlockSpec((IS, H), lambda g: (0, 0)),
        ],
        out_specs=pl.BlockSpec((_TS, IS), lambda g: (g, 0)),
        out_shape=jax.ShapeDtypeStruct((S, IS), F32),
    )(x, wg_s, wu_s)


def _sdown(hs, wd_s, routed):
    return pl.pallas_call(
        _sdown_body,
        grid=(S // _TS,),
        in_specs=[
            pl.BlockSpec((_TS, IS), lambda g: (g, 0)),
            pl.BlockSpec((H, IS), lambda g: (0, 0)),
            pl.BlockSpec((_TS, H), lambda g: (g, 0)),
        ],
        out_specs=pl.BlockSpec((_TS, H), lambda g: (g, 0)),
        out_shape=jax.ShapeDtypeStruct((S, H), F32),
    )(hs, wd_s, routed)


def kernel(hidden_states, W_gate, Wg_e, Wu_e, Wd_e, Wg_s, Wu_s, Wd_s):
    x = hidden_states.reshape(S, H)
    o_idx, o_w, o_pos, o_meta = _router(x, W_gate)
    meta = o_meta.reshape(G + 1)
    xs = _builder(x, o_pos.reshape(SK // _PB, _PB))
    hs = _sgup(x, Wg_s, Wu_s)
    ys = _grouped(meta, xs, Wg_e, Wu_e, Wd_e)
    routed = _combine(ys, o_pos.reshape(SK), o_w)
    out = _sdown(hs, Wd_s, routed)
    return out.reshape(1, S, H), o_idx.reshape(1, S, K)


# speedup vs baseline: 1.1016x; 1.0312x over previous
import functools

import jax
import jax.numpy as jnp
from jax import lax
from jax.experimental import pallas as pl
from jax.experimental.pallas import tpu as pltpu
from jax.experimental.pallas import tpu_sc as plsc

S = 2048
H = 2048
E = 8
K = 2
I = 1408
IS = 2 * I
SK = S * K
T = 256
G = SK // T + E - 1
R = G * T

_C = 512
F32 = jnp.float32
I32 = jnp.int32
BF16 = jnp.bfloat16


def _router_body(x_ref, wg_ref, idx_ref, w_ref, pos_ref, meta_ref):
    x = x_ref[...]
    logits = lax.dot_general(x, wg_ref[...], (((1,), (1,)), ((), ())),
                             preferred_element_type=F32)
    m = jnp.max(logits, axis=1, keepdims=True)
    p = jnp.exp(logits - m)
    scores = p / jnp.sum(p, axis=1, keepdims=True)

    i8 = lax.broadcasted_iota(I32, (S, E), 1).astype(F32)
    w0 = jnp.max(scores, axis=1, keepdims=True)
    idx0 = jnp.min(jnp.where(scores == w0, i8, float(E)), axis=1,
                   keepdims=True)
    masked = jnp.where(i8 == idx0, -1.0, scores)
    w1 = jnp.max(masked, axis=1, keepdims=True)
    idx1 = jnp.min(jnp.where(masked == w1, i8, float(E)), axis=1,
                   keepdims=True)

    idx_ref[...] = jnp.concatenate([idx0.astype(I32), idx1.astype(I32)],
                                   axis=1)
    w_cat = jnp.concatenate([w0, w1], axis=0)
    w_ref[...] = w_cat * jnp.ones((1, 16), F32)

    oh = jnp.concatenate([(i8 == idx0).astype(F32),
                          (i8 == idx1).astype(F32)], axis=0)

    tri = (lax.broadcasted_iota(I32, (_C, _C), 0) >=
           lax.broadcasted_iota(I32, (_C, _C), 1)).astype(F32)
    carry = jnp.zeros((1, E), F32)
    chunks = []
    for c in range(SK // _C):
        seg = oh[c * _C:(c + 1) * _C]
        incl_c = lax.dot_general(tri, seg, (((1,), (0,)), ((), ())),
                                 preferred_element_type=F32) + carry
        chunks.append(incl_c)
        carry = incl_c[_C - 1:_C, :]
    incl = jnp.concatenate(chunks, axis=0)
    rank = incl - oh

    counts = incl[SK - 1:SK, :]
    ci = counts.astype(I32)
    pc = jnp.bitwise_and(ci + (T - 1), -T)
    pcf = pc.astype(F32)
    upper = (lax.broadcasted_iota(I32, (E, E), 0) <
             lax.broadcasted_iota(I32, (E, E), 1)).astype(F32)
    aligned = lax.dot_general(pcf, upper, (((1,), (0,)), ((), ())),
                              preferred_element_type=F32)
    posf = jnp.sum(oh * (aligned + rank), axis=1, keepdims=True)
    pos_ref[...] = posf.astype(I32)

    ends = aligned + pcf
    gvals = lax.broadcasted_iota(I32, (G, 1), 0).astype(F32) * float(T)
    be = jnp.sum((gvals >= ends).astype(F32), axis=1, keepdims=True)
    eid = lax.broadcasted_iota(I32, (1, E), 1).astype(F32)
    last_e = jnp.max(jnp.where(pcf > 0, eid, 0.0), axis=1, keepdims=True)
    bec = jnp.minimum(be, last_e)
    used = jnp.sum(pcf, axis=1, keepdims=True) * (1.0 / T)
    meta_ref[...] = jnp.concatenate([used, bec], axis=0).astype(I32)


def _router(x, w_gate):
    return pl.pallas_call(
        _router_body,
        out_shape=[
            jax.ShapeDtypeStruct((S, K), I32),
            jax.ShapeDtypeStruct((SK, 16), F32),
            jax.ShapeDtypeStruct((SK, 1), I32),
            jax.ShapeDtypeStruct((G + 1, 1), I32),
        ],
    )(x, w_gate)


_NW = 32
_PPW = SK // _NW
_PB = 16


def _builder(x, pos2):
    mesh = plsc.VectorSubcoreMesh(core_axis_name="c", subcore_axis_name="s")
    nb = _PPW // _PB

    @functools.partial(
        pl.kernel,
        mesh=mesh,
        out_type=jax.ShapeDtypeStruct((R, H), F32),
        scratch_types=[
            pltpu.VMEM((nb, _PB), I32),
            pltpu.VMEM((2, _PB), I32),
            pltpu.VMEM((2, _PB, H), F32),
            pltpu.SemaphoreType.DMA,
            pltpu.SemaphoreType.DMA,
            pltpu.SemaphoreType.DMA,
            pltpu.SemaphoreType.DMA,
        ],
    )
    def body(x_hbm, pos_hbm, xs_hbm, posv, tokv, rows,
             gsem0, gsem1, ssem0, ssem1):
        wid = lax.axis_index("s") * 2 + lax.axis_index("c")
        sub = jnp.where(wid >= _NW // 2, S, 0)
        base0 = wid * _PPW
        pltpu.sync_copy(pos_hbm.at[pl.ds(wid * nb, nb)], posv)
        gsems = (gsem0, gsem1)
        ssems = (ssem0, ssem1)

        def start_gather(b):
            sl = b % 2
            for h in range(_PB // 16):
                tokv[sl, pl.ds(h * 16, 16)] = (
                    base0 - sub + b * _PB + h * 16
                    + lax.broadcasted_iota(I32, (16,), 0))
            return pltpu.async_copy(x_hbm.at[tokv.at[sl]], rows.at[sl],
                                    gsems[sl])

        ghand = [start_gather(0), None]
        shand = [None, None]
        for b in range(nb):
            sl = b % 2
            ghand[sl].wait()
            if b + 1 < nb:
                osl = (b + 1) % 2
                if shand[osl] is not None:
                    shand[osl].wait()
                ghand[osl] = start_gather(b + 1)
            shand[sl] = pltpu.async_copy(rows.at[sl], xs_hbm.at[posv.at[b]],
                                         ssems[sl])
        for hnd in shand:
            if hnd is not None:
                hnd.wait()

    return body(x, pos2)


def _gup_body(m_ref, xs_ref, wg_ref, wu_ref, h_ref):
    g = pl.program_id(0)

    @pl.when(g < m_ref[0])
    def _():
        xb = xs_ref[...].astype(BF16)
        gate = lax.dot_general(xb, wg_ref[0].astype(BF16),
                               (((1,), (1,)), ((), ())),
                               preferred_element_type=F32)
        up = lax.dot_general(xb, wu_ref[0].astype(BF16),
                             (((1,), (1,)), ((), ())),
                             preferred_element_type=F32)
        h_ref[...] = jax.nn.silu(gate) * up


def _down_body(m_ref, h_ref, wd_ref, o_ref):
    g = pl.program_id(0)

    @pl.when(g < m_ref[0])
    def _():
        o_ref[...] = lax.dot_general(h_ref[...].astype(BF16),
                                     wd_ref[0].astype(BF16),
                                     (((1,), (1,)), ((), ())),
                                     preferred_element_type=F32)


def _grouped(meta, xs, wg_e, wu_e, wd_e):
    gup_spec = pltpu.PrefetchScalarGridSpec(
        num_scalar_prefetch=1,
        grid=(G,),
        in_specs=[
            pl.BlockSpec((T, H), lambda g, m: (jnp.minimum(g, m[0] - 1), 0)),
            pl.BlockSpec((1, I, H), lambda g, m: (m[1 + g], 0, 0)),
            pl.BlockSpec((1, I, H), lambda g, m: (m[1 + g], 0, 0)),
        ],
        out_specs=pl.BlockSpec((T, I), lambda g, m: (g, 0)),
    )
    h = pl.pallas_call(
        _gup_body,
        grid_spec=gup_spec,
        out_shape=jax.ShapeDtypeStruct((R, I), F32),
    )(meta, xs, wg_e, wu_e)
    down_spec = pltpu.PrefetchScalarGridSpec(
        num_scalar_prefetch=1,
        grid=(G,),
        in_specs=[
            pl.BlockSpec((T, I), lambda g, m: (jnp.minimum(g, m[0] - 1), 0)),
            pl.BlockSpec((1, H, I), lambda g, m: (m[1 + g], 0, 0)),
        ],
        out_specs=pl.BlockSpec((T, H), lambda g, m: (g, 0)),
    )
    return pl.pallas_call(
        _down_body,
        grid_spec=down_spec,
        out_shape=jax.ShapeDtypeStruct((R, H), F32),
    )(meta, h, wd_e)


_TPW = S // _NW
_TB = 8
_CU = 4


def _combine(ys, pos, wsplat):
    mesh = plsc.VectorSubcoreMesh(core_axis_name="c", subcore_axis_name="s")
    nb = _TPW // _TB

    @functools.partial(
        pl.kernel,
        mesh=mesh,
        out_type=jax.ShapeDtypeStruct((S, H), F32),
        scratch_types=[
            pltpu.VMEM((_TPW,), I32),
            pltpu.VMEM((_TPW,), I32),
            pltpu.VMEM((_TPW, 16), F32),
            pltpu.VMEM((_TPW, 16), F32),
            pltpu.VMEM((2, 2 * _TB, H), F32),
            pltpu.VMEM((2, _TB, H), F32),
            pltpu.SemaphoreType.DMA,
            pltpu.SemaphoreType.DMA,
            pltpu.SemaphoreType.DMA,
            pltpu.SemaphoreType.DMA,
        ],
    )
    def body(ys_hbm, pos_hbm, w_hbm, out_hbm, posa, posb, wva, wvb,
             yrows, orows, gsem0, gsem1, wsem0, wsem1):
        wid = lax.axis_index("s") * 2 + lax.axis_index("c")
        base = wid * _TPW
        pltpu.sync_copy(pos_hbm.at[pl.ds(base, _TPW)], posa)
        pltpu.sync_copy(pos_hbm.at[pl.ds(S + base, _TPW)], posb)
        pltpu.sync_copy(w_hbm.at[pl.ds(base, _TPW)], wva)
        pltpu.sync_copy(w_hbm.at[pl.ds(S + base, _TPW)], wvb)
        gsems = (gsem0, gsem1)
        wsems = (wsem0, wsem1)

        def start_gather(b):
            sl = b % 2
            ha = pltpu.async_copy(ys_hbm.at[posa.at[pl.ds(b * _TB, _TB)]],
                                  yrows.at[sl, pl.ds(0, _TB)], gsems[sl])
            hb = pltpu.async_copy(ys_hbm.at[posb.at[pl.ds(b * _TB, _TB)]],
                                  yrows.at[sl, pl.ds(_TB, _TB)], gsems[sl])
            return (ha, hb)

        ghand = [start_gather(0), None]
        whand = [None, None]
        for b in range(nb):
            sl = b % 2
            t0 = base + b * _TB
            if b + 1 < nb:
                ghand[(b + 1) % 2] = start_gather(b + 1)
            wa = [wva[b * _TB + j, pl.ds(0, 16)] for j in range(_TB)]
            wb = [wvb[b * _TB + j, pl.ds(0, 16)] for j in range(_TB)]
            for hnd in ghand[sl]:
                hnd.wait()
            if whand[sl] is not None:
                whand[sl].wait()

            def cbody(c, _):
                for cc in range(_CU):
                    sl2 = pl.ds((c * _CU + cc) * 16, 16)
                    for j in range(_TB):
                        orows[sl, j, sl2] = (wa[j] * yrows[sl, j, sl2]
                                             + wb[j] * yrows[sl, _TB + j, sl2])
                return 0
            lax.fori_loop(0, H // (16 * _CU), cbody, 0)
            whand[sl] = pltpu.async_copy(orows.at[sl],
                                         out_hbm.at[pl.ds(t0, _TB)],
                                         wsems[sl])
        for hnd in whand:
            if hnd is not None:
                hnd.wait()

    return body(ys, pos, wsplat)


def _sgup_body(x_ref, wg_ref, wu_ref, h_ref):
    xb = x_ref[...].astype(BF16)
    gate = lax.dot_general(xb, wg_ref[...].astype(BF16),
                           (((1,), (1,)), ((), ())),
                           preferred_element_type=F32)
    up = lax.dot_general(xb, wu_ref[...].astype(BF16),
                         (((1,), (1,)), ((), ())),
                         preferred_element_type=F32)
    h_ref[...] = jax.nn.silu(gate) * up


def _sdown_body(h_ref, wd_ref, r_ref, o_ref):
    o_ref[...] = (lax.dot_general(h_ref[...].astype(BF16),
                                  wd_ref[...].astype(BF16),
                                  (((1,), (1,)), ((), ())),
                                  preferred_element_type=F32)
                  + r_ref[...])


_TS = 256


def _sgup(x, wg_s, wu_s):
    return pl.pallas_call(
        _sgup_body,
        grid=(S // _TS,),
        in_specs=[
            pl.BlockSpec((_TS, H), lambda g: (g, 0)),
            pl.BlockSpec((IS, H), lambda g: (0, 0)),
            pl.BlockSpec((IS, H), lambda g: (0, 0)),
        ],
        out_specs=pl.BlockSpec((_TS, IS), lambda g: (g, 0)),
        out_shape=jax.ShapeDtypeStruct((S, IS), F32),
    )(x, wg_s, wu_s)


def _sdown(hs, wd_s, routed):
    return pl.pallas_call(
        _sdown_body,
        grid=(S // _TS,),
        in_specs=[
            pl.BlockSpec((_TS, IS), lambda g: (g, 0)),
            pl.BlockSpec((H, IS), lambda g: (0, 0)),
            pl.BlockSpec((_TS, H), lambda g: (g, 0)),
        ],
        out_specs=pl.BlockSpec((_TS, H), lambda g: (g, 0)),
        out_shape=jax.ShapeDtypeStruct((S, H), F32),
    )(hs, wd_s, routed)


def kernel(hidden_states, W_gate, Wg_e, Wu_e, Wd_e, Wg_s, Wu_s, Wd_s):
    x = hidden_states.reshape(S, H)
    o_idx, o_w, o_pos, o_meta = _router(x, W_gate)
    meta = o_meta.reshape(G + 1)
    xs = _builder(x, o_pos.reshape(SK // _PB, _PB))
    hs = _sgup(x, Wg_s, Wu_s)
    ys = _grouped(meta, xs, Wg_e, Wu_e, Wd_e)
    routed = _combine(ys, o_pos.reshape(SK), o_w)
    out = _sdown(hs, Wd_s, routed)
    return out.reshape(1, S, H), o_idx.reshape(1, S, K)
